# baseline (speedup 1.0000x reference)
import functools

import jax
import jax.numpy as jnp
from jax import lax
from jax.experimental import pallas as pl
from jax.experimental.pallas import tpu as pltpu
from jax.experimental.pallas import tpu_sc as plsc

_C = 100000
_D = 128
_N = 1024
_MOM = 0.9
_TEMP = 0.15
_INV_TEMP = 1.0 / _TEMP

_CB = 10000
_NBLK = _C // _CB
_NSUB = 5
_CSUB = _CB // _NSUB


_SC_NC = 2
_SC_NS = 16
_NW = _SC_NC * _SC_NS
_BPW = _N // _NW


@functools.lru_cache(maxsize=1)
def _sc_gather_fn():
    @functools.partial(
        pl.kernel,
        mesh=plsc.VectorSubcoreMesh(core_axis_name="c", subcore_axis_name="s"),
        out_type=jax.ShapeDtypeStruct((_N, _D), jnp.float32),
        scratch_types=[
            pltpu.VMEM((_BPW,), jnp.int32),
            pltpu.VMEM((_BPW, _D), jnp.float32),
            pltpu.SemaphoreType.DMA,
        ],
    )
    def _sc_gather(table_hbm, idx_hbm, out_hbm, idx_v, rows_v, sem):
        wid = lax.axis_index("s") * _SC_NC + lax.axis_index("c")
        base = wid * _BPW
        pltpu.sync_copy(idx_hbm.at[pl.ds(base, _BPW)], idx_v)
        pltpu.async_copy(table_hbm.at[idx_v], rows_v, sem).wait()
        pltpu.sync_copy(rows_v, out_hbm.at[pl.ds(base, _BPW)])

    return _sc_gather


def _row_normalize(x):
    n = jnp.sqrt(jnp.sum(x * x, axis=1, keepdims=True))
    return x / jnp.clip(n, 1e-12)


_K = 1.4426950408889634 * _INV_TEMP


def _tc_body(feat_ref, lcol_ref, lrow_ref, cw_ref, pg_ref, protos_ref,
             out_ref, f_s, fb_s, acc_s):
    i = pl.program_id(0)

    @pl.when(i == 0)
    def _init():
        f = _row_normalize(feat_ref[...])
        f_s[...] = f
        fb_s[...] = (f * _K).astype(jnp.bfloat16)
        acc_s[...] = jnp.zeros_like(acc_s)

    fb = fb_s[...]
    for k in range(_NSUB):
        blk = protos_ref[pl.ds(k * _CSUB, _CSUB), :].astype(jnp.bfloat16)
        l2 = lax.dot_general(
            fb, blk, (((1,), (1,)), ((), ())),
            preferred_element_type=jnp.float32)
        acc_s[...] += jnp.exp2(l2.astype(jnp.bfloat16))

    @pl.when(i == _NBLK - 1)
    def _final():
        f = f_s[...]
        fb = fb_s[...]
        lcol = lcol_ref[...]
        lrow = lrow_ref[...]
        S = (lcol == lrow).astype(jnp.float32)
        ones_n = jnp.ones((_N, 1), jnp.float32)
        counts = jnp.dot(S, ones_n, preferred_element_type=jnp.float32)
        sums = jnp.dot(S, f, preferred_element_type=jnp.float32)
        mean = _row_normalize(sums / jnp.clip(counts, 1.0))
        pg = pg_ref[...]
        upd = _row_normalize(_MOM * pg + (1.0 - _MOM) * mean)
        old2 = lax.dot_general(
            fb, pg.astype(jnp.bfloat16), (((1,), (1,)), ((), ())),
            preferred_element_type=jnp.float32)
        new2 = lax.dot_general(
            f * _K, upd, (((1,), (1,)), ((), ())),
            preferred_element_type=jnp.float32)
        old_e = jnp.exp2(old2.astype(jnp.bfloat16)).astype(jnp.float32)
        ediff = jnp.exp2(new2) - old_e
        winv = 1.0 / counts
        delta = jnp.dot(ediff, winv, preferred_element_type=jnp.float32)
        acc = jnp.sum(acc_s[...].astype(jnp.float32), axis=1,
                      keepdims=True)
        z = acc + delta
        diag = jnp.sum(f * upd, axis=1, keepdims=True) * _INV_TEMP
        cw = cw_ref[...]
        nll = jnp.log(z) - diag
        num = jnp.sum(nll * cw, axis=(0, 1), keepdims=True)
        den = jnp.sum(cw, axis=(0, 1), keepdims=True)
        out_ref[...] = num / jnp.clip(den, 1e-12)


def _tc_loss(features, labels_col, labels_row, conf_w, pg, prototypes):
    return pl.pallas_call(
        _tc_body,
        grid=(_NBLK,),
        in_specs=[
            pl.BlockSpec((_N, _D), lambda i: (0, 0)),
            pl.BlockSpec((_N, 1), lambda i: (0, 0)),
            pl.BlockSpec((1, _N), lambda i: (0, 0)),
            pl.BlockSpec((_N, 1), lambda i: (0, 0)),
            pl.BlockSpec((_N, _D), lambda i: (0, 0)),
            pl.BlockSpec((_CB, _D), lambda i: (i, 0)),
        ],
        out_specs=pl.BlockSpec((1, 1), lambda i: (0, 0)),
        out_shape=jax.ShapeDtypeStruct((1, 1), jnp.float32),
        scratch_shapes=[
            pltpu.VMEM((_N, _D), jnp.float32),
            pltpu.VMEM((_N, _D), jnp.bfloat16),
            pltpu.VMEM((_N, _CSUB), jnp.bfloat16),
        ],
    )(features, labels_col, labels_row, conf_w, pg, prototypes)


def kernel(features, labels, conf_weights, prototypes):
    labels = labels.astype(jnp.int32)
    pg = _sc_gather_fn()(prototypes, labels)
    out = _tc_loss(
        features,
        labels.reshape(_N, 1),
        labels.reshape(1, _N),
        conf_weights.reshape(_N, 1),
        pg,
        prototypes,
    )
    return out[0, 0]


# speedup vs baseline: 1.0381x; 1.0381x over previous
import functools

import jax
import jax.numpy as jnp
from jax import lax
from jax.experimental import pallas as pl
from jax.experimental.pallas import tpu as pltpu
from jax.experimental.pallas import tpu_sc as plsc

_C = 100000
_D = 128
_N = 1024
_MOM = 0.9
_TEMP = 0.15
_INV_TEMP = 1.0 / _TEMP

_CB = 10000
_NBLK = _C // _CB
_NSUB = 5
_CSUB = _CB // _NSUB


_SC_NC = 2
_SC_NS = 16
_NW = _SC_NC * _SC_NS
_BPW = _N // _NW


@functools.lru_cache(maxsize=1)
def _sc_gather_fn():
    @functools.partial(
        pl.kernel,
        mesh=plsc.VectorSubcoreMesh(core_axis_name="c", subcore_axis_name="s"),
        out_type=jax.ShapeDtypeStruct((_N, _D), jnp.float32),
        scratch_types=[
            pltpu.VMEM((_BPW,), jnp.int32),
            pltpu.VMEM((_BPW, _D), jnp.float32),
            pltpu.SemaphoreType.DMA,
        ],
    )
    def _sc_gather(table_hbm, idx_hbm, out_hbm, idx_v, rows_v, sem):
        wid = lax.axis_index("s") * _SC_NC + lax.axis_index("c")
        base = wid * _BPW
        pltpu.sync_copy(idx_hbm.at[pl.ds(base, _BPW)], idx_v)
        pltpu.async_copy(table_hbm.at[idx_v], rows_v, sem).wait()
        pltpu.sync_copy(rows_v, out_hbm.at[pl.ds(base, _BPW)])

    return _sc_gather


def _row_normalize(x):
    n = jnp.sqrt(jnp.sum(x * x, axis=1, keepdims=True))
    return x / jnp.clip(n, 1e-12)


_K = 1.4426950408889634 * _INV_TEMP


def _stream_body(feat_ref, protos_ref, out_ref, fb_s, acc_s):
    i = pl.program_id(0)

    @pl.when(i == 0)
    def _init():
        f = _row_normalize(feat_ref[...])
        fb_s[...] = (f * _K).astype(jnp.bfloat16)
        acc_s[...] = jnp.zeros_like(acc_s)

    fb = fb_s[...]
    for k in range(_NSUB):
        blk = protos_ref[pl.ds(k * _CSUB, _CSUB), :].astype(jnp.bfloat16)
        l2 = lax.dot_general(
            fb, blk, (((1,), (1,)), ((), ())),
            preferred_element_type=jnp.float32)
        acc_s[...] += jnp.exp2(l2)

    @pl.when(i == _NBLK - 1)
    def _done():
        out_ref[...] = jnp.sum(acc_s[...], axis=1, keepdims=True)


def _stream_sumexp(features, prototypes):
    return pl.pallas_call(
        _stream_body,
        grid=(_NBLK,),
        in_specs=[
            pl.BlockSpec((_N, _D), lambda i: (0, 0)),
            pl.BlockSpec((_CB, _D), lambda i: (i, 0)),
        ],
        out_specs=pl.BlockSpec((_N, 1), lambda i: (0, 0)),
        out_shape=jax.ShapeDtypeStruct((_N, 1), jnp.float32),
        scratch_shapes=[
            pltpu.VMEM((_N, _D), jnp.bfloat16),
            pltpu.VMEM((_N, _CSUB), jnp.float32),
        ],
    )(features, prototypes)


def _finish_body(feat_ref, lcol_ref, lrow_ref, cw_ref, pg_ref, acc_ref,
                 out_ref):
    f = _row_normalize(feat_ref[...])
    fb = (f * _K).astype(jnp.bfloat16)
    lcol = lcol_ref[...]
    lrow = lrow_ref[...]
    S = (lcol == lrow).astype(jnp.float32)
    ones_n = jnp.ones((_N, 1), jnp.float32)
    counts = jnp.dot(S, ones_n, preferred_element_type=jnp.float32)
    sums = jnp.dot(S, f, preferred_element_type=jnp.float32)
    mean = _row_normalize(sums / jnp.clip(counts, 1.0))
    pg = pg_ref[...]
    upd = _row_normalize(_MOM * pg + (1.0 - _MOM) * mean)
    old2 = lax.dot_general(
        fb, pg.astype(jnp.bfloat16), (((1,), (1,)), ((), ())),
        preferred_element_type=jnp.float32)
    new2 = lax.dot_general(
        f * _K, upd, (((1,), (1,)), ((), ())),
        preferred_element_type=jnp.float32)
    ediff = jnp.exp2(new2) - jnp.exp2(old2)
    winv = 1.0 / counts
    delta = jnp.dot(ediff, winv, preferred_element_type=jnp.float32)
    z = acc_ref[...] + delta
    diag = jnp.sum(f * upd, axis=1, keepdims=True) * _INV_TEMP
    cw = cw_ref[...]
    nll = jnp.log(z) - diag
    num = jnp.sum(nll * cw, axis=(0, 1), keepdims=True)
    den = jnp.sum(cw, axis=(0, 1), keepdims=True)
    out_ref[...] = num / jnp.clip(den, 1e-12)


def _finish(features, labels_col, labels_row, conf_w, pg, acc):
    return pl.pallas_call(
        _finish_body,
        out_shape=jax.ShapeDtypeStruct((1, 1), jnp.float32),
    )(features, labels_col, labels_row, conf_w, pg, acc)


def kernel(features, labels, conf_weights, prototypes):
    labels = labels.astype(jnp.int32)
    pg = _sc_gather_fn()(prototypes, labels)
    acc = _stream_sumexp(features, prototypes)
    out = _finish(
        features,
        labels.reshape(_N, 1),
        labels.reshape(1, _N),
        conf_weights.reshape(_N, 1),
        pg,
        acc,
    )
    return out[0, 0]
